# baseline (speedup 1.0000x reference)
import functools

import jax
import jax.numpy as jnp
from jax import lax
from jax.experimental import pallas as pl
from jax.experimental.pallas import tpu as pltpu

B = 8
C = 256
H = 64
W = 64
HW = H * W
OUT = 65
E = 4
CELL = 8
EPAD = 128


def _fused(x_ref, wg_ref, bg_ref, we_ref, bet_ref, gam_ref, bta_ref,
           outf_ref, prob_ref, loss_ref,
           acc_ref, psum_ref, psumsq_ref, cnt_ref):
    ph = pl.program_id(0)
    b = pl.program_id(1)

    @pl.when(ph == 0)
    def _phase_a():
        xb = jnp.maximum(x_ref[0], 0.0)
        pooled = jnp.sum(xb, axis=1, keepdims=True) / HW
        logits = lax.dot_general(pooled, wg_ref[...],
                                 (((0,), (0,)), ((), ())),
                                 preferred_element_type=jnp.float32)
        logits = logits + bg_ref[...]
        lanes = lax.broadcasted_iota(jnp.int32, (1, EPAD), 1)
        mx = jnp.max(logits, axis=1, keepdims=True)
        cand = jnp.where(logits >= mx, lanes, EPAD)
        eid = jnp.min(cand, axis=1, keepdims=True)
        oh = jnp.where(lanes == eid, 1.0, 0.0)

        emask3 = (lax.broadcasted_iota(jnp.int32, (E, 1, 1), 0)
                  == eid.reshape(1, 1, 1)).astype(jnp.float32)
        wsel = jnp.sum(we_ref[...] * emask3, axis=0)
        bmask = (lax.broadcasted_iota(jnp.int32, (1, E), 1)
                 == eid).astype(jnp.float32)
        bsel = jnp.sum(bet_ref[...] * bmask, axis=1, keepdims=True)

        res = lax.dot_general(wsel.astype(jnp.bfloat16),
                              xb.astype(jnp.bfloat16),
                              (((0,), (0,)), ((), ())),
                              preferred_element_type=jnp.float32)
        res = res + bsel
        acc_ref[b] = res
        part = jnp.sum(res, axis=1, keepdims=True)
        partsq = jnp.sum(res * res, axis=1, keepdims=True)

        @pl.when(b == 0)
        def _():
            psum_ref[...] = part
            psumsq_ref[...] = partsq
            cnt_ref[...] = oh

        @pl.when(b > 0)
        def _():
            psum_ref[...] += part
            psumsq_ref[...] += partsq
            cnt_ref[...] += oh

    @pl.when(ph == 1)
    def _phase_b():
        n = float(B * HW)
        mean = psum_ref[...] / n
        var = psumsq_ref[...] / n - mean * mean
        rstd = lax.rsqrt(var + 1e-5)
        o = (acc_ref[b] - mean) * (rstd * gam_ref[...]) + bta_ref[...]
        outf_ref[0] = o

        mx = jnp.max(o, axis=0, keepdims=True)
        ex = jnp.exp(o - mx)
        sm = ex / jnp.sum(ex, axis=0, keepdims=True)
        p = sm[:CELL * CELL, :].astype(jnp.bfloat16)
        p4 = p.reshape(CELL, CELL, H, W)
        pt = jnp.transpose(p4, (2, 0, 3, 1))
        prob_ref[0] = pt.reshape(H * CELL, W * CELL).astype(jnp.float32)

        lanes = lax.broadcasted_iota(jnp.int32, (1, EPAD), 1)
        valid = lanes < E
        u = jnp.where(valid, cnt_ref[...] / B + 1e-6, 0.0)
        u = u / jnp.sum(u, axis=1, keepdims=True)
        usafe = jnp.where(valid, u, 1.0)
        term = u * (jnp.log(usafe) + jnp.log(float(E)))
        loss_ref[...] = jnp.broadcast_to(
            jnp.sum(term, axis=1, keepdims=True), (1, EPAD))


@functools.partial(jax.jit, static_argnames=("interpret",))
def kernel(x, We, be, Wg, bg, gamma, beta, interpret=False):
    x_r = x.reshape(B, C, HW)
    wg_p = jnp.zeros((C, EPAD), jnp.float32).at[:, :E].set(Wg)
    bg_p = jnp.full((1, EPAD), -1e30, jnp.float32).at[0, :E].set(bg)
    be_t = be.T
    gam = gamma.reshape(OUT, 1)
    bta = beta.reshape(OUT, 1)

    outf, prob, loss_arr = pl.pallas_call(
        _fused,
        grid=(2, B),
        in_specs=[
            pl.BlockSpec((1, C, HW),
                         lambda i, j: (jnp.where(i == 0, j, 0), 0, 0)),
            pl.BlockSpec((C, EPAD), lambda i, j: (0, 0)),
            pl.BlockSpec((1, EPAD), lambda i, j: (0, 0)),
            pl.BlockSpec((E, C, OUT), lambda i, j: (0, 0, 0)),
            pl.BlockSpec((OUT, E), lambda i, j: (0, 0)),
            pl.BlockSpec((OUT, 1), lambda i, j: (0, 0)),
            pl.BlockSpec((OUT, 1), lambda i, j: (0, 0)),
        ],
        out_specs=[
            pl.BlockSpec((1, OUT, HW), lambda i, j: (j, 0, 0)),
            pl.BlockSpec((1, H * CELL, W * CELL), lambda i, j: (j, 0, 0)),
            pl.BlockSpec((1, EPAD), lambda i, j: (0, 0)),
        ],
        out_shape=[
            jax.ShapeDtypeStruct((B, OUT, HW), jnp.float32),
            jax.ShapeDtypeStruct((B, H * CELL, W * CELL), jnp.float32),
            jax.ShapeDtypeStruct((1, EPAD), jnp.float32),
        ],
        scratch_shapes=[
            pltpu.VMEM((B, OUT, HW), jnp.float32),
            pltpu.VMEM((OUT, 1), jnp.float32),
            pltpu.VMEM((OUT, 1), jnp.float32),
            pltpu.VMEM((1, EPAD), jnp.float32),
        ],
        interpret=interpret,
    )(x_r, wg_p, bg_p, We, be_t, gam, bta)

    out = outf.reshape(B, OUT, H, W)
    loss = loss_arr[0, 0]
    return (out, prob, loss)


# speedup vs baseline: 1.0126x; 1.0126x over previous
import functools

import jax
import jax.numpy as jnp
from jax import lax
from jax.experimental import pallas as pl
from jax.experimental.pallas import tpu as pltpu

B = 8
C = 256
H = 64
W = 64
HW = H * W
OUT = 65
E = 4
CELL = 8
EPAD = 128


def _pass_a(x_ref, wg_ref, bg_ref, we_ref, bet_ref,
            out_ref, psum_ref, psumsq_ref, oh_ref):
    xb = jnp.maximum(x_ref[0], 0.0)
    pooled = jnp.sum(xb, axis=1, keepdims=True) / HW
    logits = lax.dot_general(pooled, wg_ref[...],
                             (((0,), (0,)), ((), ())),
                             preferred_element_type=jnp.float32)
    logits = logits + bg_ref[...]
    lanes = lax.broadcasted_iota(jnp.int32, (1, EPAD), 1)
    mx = jnp.max(logits, axis=1, keepdims=True)
    cand = jnp.where(logits >= mx, lanes, EPAD)
    eid = jnp.min(cand, axis=1, keepdims=True)
    oh_ref[0] = jnp.where(lanes == eid, 1.0, 0.0)

    emask3 = (lax.broadcasted_iota(jnp.int32, (E, 1, 1), 0)
              == eid.reshape(1, 1, 1)).astype(jnp.float32)
    wsel = jnp.sum(we_ref[...] * emask3, axis=0)
    bmask = (lax.broadcasted_iota(jnp.int32, (1, E), 1)
             == eid).astype(jnp.float32)
    bsel = jnp.sum(bet_ref[...] * bmask, axis=1, keepdims=True)

    res = lax.dot_general(wsel.astype(jnp.bfloat16), xb.astype(jnp.bfloat16),
                          (((0,), (0,)), ((), ())),
                          preferred_element_type=jnp.float32)
    res = res + bsel
    out_ref[0] = res
    psum_ref[0] = jnp.sum(res, axis=1, keepdims=True)
    psumsq_ref[0] = jnp.sum(res * res, axis=1, keepdims=True)


def _pass_b(out_ref, psum_ref, psumsq_ref, oh_ref, gam_ref, bet_ref,
            outf_ref, prob_ref, loss_ref):
    n = float(B * HW)
    tot = jnp.sum(psum_ref[...], axis=0)
    totsq = jnp.sum(psumsq_ref[...], axis=0)
    mean = tot / n
    var = totsq / n - mean * mean
    rstd = lax.rsqrt(var + 1e-5)
    o = (out_ref[0] - mean) * rstd
    o = o * gam_ref[...] + bet_ref[...]
    outf_ref[0] = o

    mx = jnp.max(o, axis=0, keepdims=True)
    ex = jnp.exp(o - mx)
    sm = ex / jnp.sum(ex, axis=0, keepdims=True)
    p = sm[:CELL * CELL, :].astype(jnp.bfloat16)
    p4 = p.reshape(CELL, CELL, H, W)
    pt = jnp.transpose(p4, (2, 0, 3, 1))
    prob_ref[0] = pt.reshape(H * CELL, W * CELL).astype(jnp.float32)

    lanes = lax.broadcasted_iota(jnp.int32, (1, EPAD), 1)
    valid = lanes < E
    counts = jnp.sum(oh_ref[...], axis=0)
    u = jnp.where(valid, counts / B + 1e-6, 0.0)
    u = u / jnp.sum(u, axis=1, keepdims=True)
    usafe = jnp.where(valid, u, 1.0)
    term = u * (jnp.log(usafe) + jnp.log(float(E)))
    loss_ref[0] = jnp.broadcast_to(
        jnp.sum(term, axis=1, keepdims=True), (1, EPAD))


@functools.partial(jax.jit, static_argnames=("interpret",))
def kernel(x, We, be, Wg, bg, gamma, beta, interpret=False):
    x_r = x.reshape(B, C, HW)
    wg_p = jnp.zeros((C, EPAD), jnp.float32).at[:, :E].set(Wg)
    bg_p = jnp.full((1, EPAD), -1e30, jnp.float32).at[0, :E].set(bg)
    be_t = be.T
    gam = gamma.reshape(OUT, 1)
    bet = beta.reshape(OUT, 1)

    out_pre, psum, psumsq, oh = pl.pallas_call(
        _pass_a,
        grid=(B,),
        in_specs=[
            pl.BlockSpec((1, C, HW), lambda i: (i, 0, 0)),
            pl.BlockSpec((C, EPAD), lambda i: (0, 0)),
            pl.BlockSpec((1, EPAD), lambda i: (0, 0)),
            pl.BlockSpec((E, C, OUT), lambda i: (0, 0, 0)),
            pl.BlockSpec((OUT, E), lambda i: (0, 0)),
        ],
        out_specs=[
            pl.BlockSpec((1, OUT, HW), lambda i: (i, 0, 0)),
            pl.BlockSpec((1, OUT, 1), lambda i: (i, 0, 0)),
            pl.BlockSpec((1, OUT, 1), lambda i: (i, 0, 0)),
            pl.BlockSpec((1, 1, EPAD), lambda i: (i, 0, 0)),
        ],
        out_shape=[
            jax.ShapeDtypeStruct((B, OUT, HW), jnp.float32),
            jax.ShapeDtypeStruct((B, OUT, 1), jnp.float32),
            jax.ShapeDtypeStruct((B, OUT, 1), jnp.float32),
            jax.ShapeDtypeStruct((B, 1, EPAD), jnp.float32),
        ],
        compiler_params=pltpu.CompilerParams(
            dimension_semantics=("parallel",)),
        interpret=interpret,
    )(x_r, wg_p, bg_p, We, be_t)

    outf, prob, loss_arr = pl.pallas_call(
        _pass_b,
        grid=(B,),
        in_specs=[
            pl.BlockSpec((1, OUT, HW), lambda i: (i, 0, 0)),
            pl.BlockSpec((B, OUT, 1), lambda i: (0, 0, 0)),
            pl.BlockSpec((B, OUT, 1), lambda i: (0, 0, 0)),
            pl.BlockSpec((B, 1, EPAD), lambda i: (0, 0, 0)),
            pl.BlockSpec((OUT, 1), lambda i: (0, 0)),
            pl.BlockSpec((OUT, 1), lambda i: (0, 0)),
        ],
        out_specs=[
            pl.BlockSpec((1, OUT, HW), lambda i: (i, 0, 0)),
            pl.BlockSpec((1, H * CELL, W * CELL), lambda i: (i, 0, 0)),
            pl.BlockSpec((1, 1, EPAD), lambda i: (i, 0, 0)),
        ],
        out_shape=[
            jax.ShapeDtypeStruct((B, OUT, HW), jnp.float32),
            jax.ShapeDtypeStruct((B, H * CELL, W * CELL), jnp.float32),
            jax.ShapeDtypeStruct((B, 1, EPAD), jnp.float32),
        ],
        compiler_params=pltpu.CompilerParams(
            dimension_semantics=("parallel",)),
        interpret=interpret,
    )(out_pre, psum, psumsq, oh, gam, bet)

    out = outf.reshape(B, OUT, H, W)
    loss = loss_arr[0, 0, 0]
    return (out, prob, loss)


# speedup vs baseline: 1.0198x; 1.0071x over previous
import functools

import jax
import jax.numpy as jnp
from jax import lax
from jax.experimental import pallas as pl
from jax.experimental.pallas import tpu as pltpu

B = 8
C = 256
H = 64
W = 64
HW = H * W
OUT = 65
E = 4
CELL = 8
EPAD = 128


def _pass_a(x_ref, wg_ref, bg_ref, we_ref, bet_ref,
            out_ref, psum_ref, psumsq_ref, oh_ref):
    xb = jnp.maximum(x_ref[0], 0.0)
    pooled = jnp.sum(xb, axis=1, keepdims=True) / HW
    logits = lax.dot_general(pooled, wg_ref[...],
                             (((0,), (0,)), ((), ())),
                             preferred_element_type=jnp.float32)
    logits = logits + bg_ref[...]
    lanes = lax.broadcasted_iota(jnp.int32, (1, EPAD), 1)
    mx = jnp.max(logits, axis=1, keepdims=True)
    cand = jnp.where(logits >= mx, lanes, EPAD)
    eid = jnp.min(cand, axis=1, keepdims=True)
    oh_ref[0] = jnp.where(lanes == eid, 1.0, 0.0)

    emask3 = (lax.broadcasted_iota(jnp.int32, (E, 1, 1), 0)
              == eid.reshape(1, 1, 1)).astype(jnp.float32)
    wsel = jnp.sum(we_ref[...] * emask3, axis=0)
    bmask = (lax.broadcasted_iota(jnp.int32, (1, E), 1)
             == eid).astype(jnp.float32)
    bsel = jnp.sum(bet_ref[...] * bmask, axis=1, keepdims=True)

    res = lax.dot_general(wsel.astype(jnp.bfloat16), xb.astype(jnp.bfloat16),
                          (((0,), (0,)), ((), ())),
                          preferred_element_type=jnp.float32)
    res = res + bsel
    out_ref[0] = res.astype(jnp.bfloat16)
    psum_ref[0] = jnp.sum(res, axis=1, keepdims=True)
    psumsq_ref[0] = jnp.sum(res * res, axis=1, keepdims=True)


def _pass_b(out_ref, psum_ref, psumsq_ref, oh_ref, gam_ref, bet_ref,
            outf_ref, prob_ref, loss_ref):
    n = float(B * HW)
    tot = jnp.sum(psum_ref[...], axis=0)
    totsq = jnp.sum(psumsq_ref[...], axis=0)
    mean = tot / n
    var = totsq / n - mean * mean
    rstd = lax.rsqrt(var + 1e-5)
    o = (out_ref[0].astype(jnp.float32) - mean) * rstd
    o = o * gam_ref[...] + bet_ref[...]
    outf_ref[0] = o

    mx = jnp.max(o, axis=0, keepdims=True)
    ex = jnp.exp(o - mx)
    sm = ex / jnp.sum(ex, axis=0, keepdims=True)
    p = sm[:CELL * CELL, :].astype(jnp.bfloat16)
    p4 = p.reshape(CELL, CELL, H, W)
    pt = jnp.transpose(p4, (2, 0, 3, 1))
    prob_ref[0] = pt.reshape(H * CELL, W * CELL).astype(jnp.float32)

    lanes = lax.broadcasted_iota(jnp.int32, (1, EPAD), 1)
    valid = lanes < E
    counts = jnp.sum(oh_ref[...], axis=0)
    u = jnp.where(valid, counts / B + 1e-6, 0.0)
    u = u / jnp.sum(u, axis=1, keepdims=True)
    usafe = jnp.where(valid, u, 1.0)
    term = u * (jnp.log(usafe) + jnp.log(float(E)))
    loss_ref[0] = jnp.broadcast_to(
        jnp.sum(term, axis=1, keepdims=True), (1, EPAD))


@functools.partial(jax.jit, static_argnames=("interpret",))
def kernel(x, We, be, Wg, bg, gamma, beta, interpret=False):
    x_r = x.reshape(B, C, HW)
    wg_p = jnp.zeros((C, EPAD), jnp.float32).at[:, :E].set(Wg)
    bg_p = jnp.full((1, EPAD), -1e30, jnp.float32).at[0, :E].set(bg)
    be_t = be.T
    gam = gamma.reshape(OUT, 1)
    bet = beta.reshape(OUT, 1)

    out_pre, psum, psumsq, oh = pl.pallas_call(
        _pass_a,
        grid=(B,),
        in_specs=[
            pl.BlockSpec((1, C, HW), lambda i: (i, 0, 0)),
            pl.BlockSpec((C, EPAD), lambda i: (0, 0)),
            pl.BlockSpec((1, EPAD), lambda i: (0, 0)),
            pl.BlockSpec((E, C, OUT), lambda i: (0, 0, 0)),
            pl.BlockSpec((OUT, E), lambda i: (0, 0)),
        ],
        out_specs=[
            pl.BlockSpec((1, OUT, HW), lambda i: (i, 0, 0)),
            pl.BlockSpec((1, OUT, 1), lambda i: (i, 0, 0)),
            pl.BlockSpec((1, OUT, 1), lambda i: (i, 0, 0)),
            pl.BlockSpec((1, 1, EPAD), lambda i: (i, 0, 0)),
        ],
        out_shape=[
            jax.ShapeDtypeStruct((B, OUT, HW), jnp.bfloat16),
            jax.ShapeDtypeStruct((B, OUT, 1), jnp.float32),
            jax.ShapeDtypeStruct((B, OUT, 1), jnp.float32),
            jax.ShapeDtypeStruct((B, 1, EPAD), jnp.float32),
        ],
        compiler_params=pltpu.CompilerParams(
            dimension_semantics=("parallel",)),
        interpret=interpret,
    )(x_r, wg_p, bg_p, We, be_t)

    outf, prob, loss_arr = pl.pallas_call(
        _pass_b,
        grid=(B,),
        in_specs=[
            pl.BlockSpec((1, OUT, HW), lambda i: (i, 0, 0)),
            pl.BlockSpec((B, OUT, 1), lambda i: (0, 0, 0)),
            pl.BlockSpec((B, OUT, 1), lambda i: (0, 0, 0)),
            pl.BlockSpec((B, 1, EPAD), lambda i: (0, 0, 0)),
            pl.BlockSpec((OUT, 1), lambda i: (0, 0)),
            pl.BlockSpec((OUT, 1), lambda i: (0, 0)),
        ],
        out_specs=[
            pl.BlockSpec((1, OUT, HW), lambda i: (i, 0, 0)),
            pl.BlockSpec((1, H * CELL, W * CELL), lambda i: (i, 0, 0)),
            pl.BlockSpec((1, 1, EPAD), lambda i: (i, 0, 0)),
        ],
        out_shape=[
            jax.ShapeDtypeStruct((B, OUT, HW), jnp.float32),
            jax.ShapeDtypeStruct((B, H * CELL, W * CELL), jnp.float32),
            jax.ShapeDtypeStruct((B, 1, EPAD), jnp.float32),
        ],
        compiler_params=pltpu.CompilerParams(
            dimension_semantics=("parallel",)),
        interpret=interpret,
    )(out_pre, psum, psumsq, oh, gam, bet)

    out = outf.reshape(B, OUT, H, W)
    loss = loss_arr[0, 0, 0]
    return (out, prob, loss)
